# baseline (speedup 1.0000x reference)
import functools

import jax
import jax.numpy as jnp
from jax import lax
from jax.experimental import pallas as pl
from jax.experimental.pallas import tpu as pltpu
from jax.experimental.pallas import tpu_sc as plsc

D = 128
DG = 8
NC = 2
NS = 16
NW = NC * NS
CH = 80
NB = 4


def _sc_aggregate(x, src, dst, ones_rows, zf, zd):
    epw = src.shape[0] // NW
    nch = epw // CH
    NP = zf.shape[0] * NS
    rpt = NP // NS

    mesh = plsc.VectorSubcoreMesh(
        core_axis_name="c", subcore_axis_name="s", num_cores=NC, num_subcores=NS
    )

    @functools.partial(
        pl.kernel,
        out_type=(
            jax.ShapeDtypeStruct((NC, NP, D), jnp.float32),
            jax.ShapeDtypeStruct((NC, NP, DG), jnp.float32),
        ),
        mesh=mesh,
        scratch_types=[
            pltpu.VMEM_SHARED((NP, D), jnp.float32),
            pltpu.VMEM_SHARED((NP, DG), jnp.float32),
            pltpu.VMEM((CH, DG), jnp.float32),
        ]
        + [pltpu.VMEM((CH,), jnp.int32) for _ in range(NB)]
        + [pltpu.VMEM((CH,), jnp.int32) for _ in range(NB)]
        + [pltpu.VMEM((CH, D), jnp.float32) for _ in range(NB)]
        + [pltpu.SemaphoreType.DMA for _ in range(5 * NB)],
        compiler_params=pltpu.CompilerParams(use_tc_tiling_on_sc=False),
    )
    def agg(x_hbm, src_hbm, dst_hbm, ones_hbm, zf_hbm, zd_hbm, of_hbm, od_hbm,
            facc, dacc, ones_v, *rest):
        sbuf = rest[:NB]
        dbuf = rest[NB:2 * NB]
        rows = rest[2 * NB:3 * NB]
        isem = rest[3 * NB:4 * NB]
        jsem = rest[4 * NB:5 * NB]
        gsem = rest[5 * NB:6 * NB]
        fsem = rest[6 * NB:7 * NB]
        dsem = rest[7 * NB:8 * NB]
        c = lax.axis_index("c")
        s = lax.axis_index("s")
        base = (c * NS + s) * epw
        r0 = s * rpt
        pltpu.sync_copy(zf_hbm, facc.at[pl.ds(r0, rpt)])
        pltpu.sync_copy(zd_hbm, dacc.at[pl.ds(r0, rpt)])
        pltpu.sync_copy(ones_hbm, ones_v)
        for b in range(NB):
            pltpu.sync_copy(src_hbm.at[pl.ds(base + b * CH, CH)], sbuf[b])
        pltpu.sync_copy(dst_hbm.at[pl.ds(base, CH)], dbuf[0])
        for b in range(1, NB - 1):
            pltpu.async_copy(dst_hbm.at[pl.ds(base + b * CH, CH)], dbuf[b],
                             jsem[b])
        for b in range(NB - 1):
            pltpu.async_copy(x_hbm.at[sbuf[b]], rows[b], gsem[b])
        plsc.subcore_barrier()

        def start_sfetch(b, i):
            pltpu.async_copy(src_hbm.at[pl.ds(base + i * CH, CH)], sbuf[b],
                             isem[b])

        def wait_sfetch(b, i):
            pltpu.make_async_copy(src_hbm.at[pl.ds(base + i * CH, CH)],
                                  sbuf[b], isem[b]).wait()

        def start_dfetch(b, i):
            pltpu.async_copy(dst_hbm.at[pl.ds(base + i * CH, CH)], dbuf[b],
                             jsem[b])

        def wait_dfetch(b, i):
            pltpu.make_async_copy(dst_hbm.at[pl.ds(base + i * CH, CH)],
                                  dbuf[b], jsem[b]).wait()

        def wait_gather(b):
            pltpu.make_async_copy(x_hbm.at[sbuf[b]], rows[b], gsem[b]).wait()

        def start_scatter(b):
            pltpu.async_copy(rows[b], facc.at[dbuf[b]], fsem[b], add=True)
            pltpu.async_copy(ones_v, dacc.at[dbuf[b]], dsem[b], add=True)

        def wait_scatter(b):
            pltpu.make_async_copy(rows[b], facc.at[dbuf[b]], fsem[b]).wait()
            pltpu.make_async_copy(ones_v, dacc.at[dbuf[b]], dsem[b]).wait()

        wait_gather(0)
        start_scatter(0)
        start_sfetch(0, NB)
        start_dfetch(NB - 1, NB - 1)
        pltpu.async_copy(x_hbm.at[sbuf[NB - 1]], rows[NB - 1], gsem[NB - 1])

        def sub(i, b):
            wait_gather(b)
            wait_dfetch(b, i)
            start_scatter(b)

            @pl.when(i + NB < nch)
            def _():
                start_sfetch(b, i + NB)

            @pl.when(i + NB - 1 < nch)
            def _():
                bg = (b - 1) % NB
                wait_sfetch(bg, i + NB - 1)
                wait_scatter(bg)
                start_dfetch(bg, i + NB - 1)
                pltpu.async_copy(x_hbm.at[sbuf[bg]], rows[bg], gsem[bg])

            return None

        def outer(k, carry):
            i = NB * k + 1
            sub(i, 1)
            sub(i + 1, 2)
            sub(i + 2, 3)
            sub(i + 3, 0)
            return carry

        lax.fori_loop(0, (nch - 1) // NB, outer, 0)
        for i in range(nch - NB, nch):
            wait_scatter(i % NB)
        plsc.subcore_barrier()
        pltpu.sync_copy(facc.at[pl.ds(r0, rpt)], of_hbm.at[c, pl.ds(r0, rpt)])
        pltpu.sync_copy(dacc.at[pl.ds(r0, rpt)], od_hbm.at[c, pl.ds(r0, rpt)])

    return agg(x, src, dst, ones_rows, zf, zd)


def _tc_combine(feats, degs, x, W_self, b_self, W_neigh, b_neigh, bias):
    N = x.shape[0]
    BL = 1000
    grid = (N // BL,)

    def body(f_ref, d_ref, x_ref, ws_ref, bs_ref, wn_ref, bn_ref, b_ref, o_ref):
        p = f_ref[0] + f_ref[1]
        d = d_ref[0] + d_ref[1]
        deg = jnp.sum(d, axis=1, keepdims=True)
        hn = p / jnp.maximum(deg, 1.0)
        h_self = lax.dot_general(
            x_ref[...], ws_ref[...], (((1,), (1,)), ((), ())),
            preferred_element_type=jnp.float32,
        ) + bs_ref[...]
        h_neigh = lax.dot_general(
            hn, wn_ref[...], (((1,), (1,)), ((), ())),
            preferred_element_type=jnp.float32,
        ) + bn_ref[...]
        o_ref[...] = (h_self + h_neigh) * 0.5 + b_ref[...]

    blk = lambda shape: pl.BlockSpec(shape, lambda i: (0,) * len(shape))
    return pl.pallas_call(
        body,
        grid=grid,
        in_specs=[
            pl.BlockSpec((NC, BL, D), lambda i: (0, i, 0)),
            pl.BlockSpec((NC, BL, DG), lambda i: (0, i, 0)),
            pl.BlockSpec((BL, D), lambda i: (i, 0)),
            blk((D, D)),
            blk((1, D)),
            blk((D, D)),
            blk((1, D)),
            blk((1, D)),
        ],
        out_specs=pl.BlockSpec((BL, D), lambda i: (i, 0)),
        out_shape=jax.ShapeDtypeStruct((N, D), jnp.float32),
    )(feats, degs, x, W_self, b_self.reshape(1, D), W_neigh,
      b_neigh.reshape(1, D), bias.reshape(1, D))


def kernel(batch_input_feats, batch_input_labels, batch_input_labels_ori,
           batch_cent_feats, batch_cent_labels, batch_cent_labels_ori,
           W_self, b_self, W_neigh, b_neigh, bias, edge_index):
    x = batch_input_feats
    N = x.shape[0]
    src = edge_index[0]
    dst = edge_index[1]
    np_rows = ((N + 8 * NS - 1) // (8 * NS)) * 8 * NS
    zf = jnp.zeros((np_rows // NS, D), jnp.float32)
    zd = jnp.zeros((np_rows // NS, DG), jnp.float32)
    ones_rows = jnp.full((CH, DG), 1.0 / DG, jnp.float32)
    feats, degs = _sc_aggregate(x, src, dst, ones_rows, zf, zd)
    return _tc_combine(feats, degs, x, W_self, b_self, W_neigh, b_neigh, bias)
